# Initial kernel scaffold; baseline (speedup 1.0000x reference)
import jax
import jax.numpy as jnp
from jax.experimental import pallas as pl


def kernel(x, edge_attr, edge_index):
    raise NotImplementedError("write your pallas kernel here")


# speedup vs baseline: 3.5465x; 3.5465x over previous
import jax
import jax.numpy as jnp
from jax import lax
from jax.experimental import pallas as pl
from jax.experimental.pallas import tpu as pltpu
from jax.experimental.pallas import tpu_sc as plsc

N = 10000
E = 320000
D = 128

NC = 2
NS = 16
E_PER_TILE = E // (NC * NS)
K = 80
CHUNKS = E_PER_TILE // K
ROWS_PER_TILE = 624
ROWS_TAIL = N - NS * ROWS_PER_TILE


def _sc_body(x_hbm, src_hbm, dst_hbm, attr_hbm, zeros_hbm, out_hbm,
             sidx_v, didx_v, rows_v, attr_v, acc, sem):
    c = lax.axis_index("c")
    s = lax.axis_index("s")
    tile_base = (c * NS + s) * E_PER_TILE

    r0 = s * ROWS_PER_TILE
    pltpu.sync_copy(zeros_hbm.at[pl.ds(r0, ROWS_PER_TILE)],
                    acc.at[pl.ds(r0, ROWS_PER_TILE)])

    @pl.when(s == NS - 1)
    def _zero_tail():
        pltpu.sync_copy(zeros_hbm.at[pl.ds(NS * ROWS_PER_TILE, ROWS_TAIL)],
                        acc.at[pl.ds(NS * ROWS_PER_TILE, ROWS_TAIL)])

    plsc.subcore_barrier()

    @pl.loop(0, CHUNKS)
    def _chunk(j):
        eb = pl.multiple_of(tile_base + j * K, 8)
        pltpu.sync_copy(src_hbm.at[pl.ds(eb, K)], sidx_v)
        pltpu.sync_copy(dst_hbm.at[pl.ds(eb, K)], didx_v)
        pltpu.async_copy(x_hbm.at[sidx_v], rows_v, sem).wait()
        pltpu.sync_copy(attr_hbm.at[pl.ds(eb, K)], attr_v)
        pltpu.sync_copy(rows_v, acc.at[didx_v], add=True)
        pltpu.sync_copy(attr_v, acc.at[didx_v], add=True)

    plsc.subcore_barrier()
    pltpu.sync_copy(acc.at[pl.ds(r0, ROWS_PER_TILE)],
                    out_hbm.at[c, pl.ds(r0, ROWS_PER_TILE)])

    @pl.when(s == NS - 1)
    def _write_tail():
        pltpu.sync_copy(acc.at[pl.ds(NS * ROWS_PER_TILE, ROWS_TAIL)],
                        out_hbm.at[c, pl.ds(NS * ROWS_PER_TILE, ROWS_TAIL)])


_sc_scatter = pl.kernel(
    _sc_body,
    out_type=jax.ShapeDtypeStruct((NC, N, D), jnp.float32),
    mesh=plsc.VectorSubcoreMesh(core_axis_name="c", subcore_axis_name="s"),
    scratch_types=[
        pltpu.VMEM((K,), jnp.int32),
        pltpu.VMEM((K,), jnp.int32),
        pltpu.VMEM((K, D), jnp.float32),
        pltpu.VMEM((K, D), jnp.float32),
        pltpu.VMEM_SHARED((N, D), jnp.float32),
        pltpu.SemaphoreType.DMA,
    ],
)


def _combine_body(x_ref, p0_ref, p1_ref, o_ref):
    o_ref[...] = x_ref[...] + p0_ref[...] + p1_ref[...]


_combine = pl.pallas_call(
    _combine_body,
    grid=(10,),
    in_specs=[pl.BlockSpec((1000, D), lambda i: (i, 0))] * 3,
    out_specs=pl.BlockSpec((1000, D), lambda i: (i, 0)),
    out_shape=jax.ShapeDtypeStruct((N, D), jnp.float32),
)


@jax.jit
def kernel(x, edge_attr, edge_index):
    src = edge_index[0]
    dst = edge_index[1]
    zeros = jnp.zeros_like(x)
    partials = _sc_scatter(x, src, dst, edge_attr, zeros)
    return _combine(x, partials[0], partials[1])


# speedup vs baseline: 5.5509x; 1.5652x over previous
import jax
import jax.numpy as jnp
from jax import lax
from jax.experimental import pallas as pl
from jax.experimental.pallas import tpu as pltpu
from jax.experimental.pallas import tpu_sc as plsc

N = 10000
E = 320000
D = 128

NC = 2
NS = 16
NW = NC * NS
K = 64
CHUNKS = 156
EXTRA_BASE = NW * CHUNKS * K
EXTRA_TILES = (E - EXTRA_BASE) // K
ROWS_PER_TILE = 624
ROWS_TAIL = N - NS * ROWS_PER_TILE


def _sc_body(x_hbm, src_hbm, dst_hbm, attr_hbm, zeros_hbm, out_hbm,
             sidx, didx, rows, attr, acc,
             sem_in, sem_s):
    c = lax.axis_index("c")
    s = lax.axis_index("s")
    tid = c * NS + s
    tile_base = tid * (CHUNKS * K)

    r0 = s * ROWS_PER_TILE
    pltpu.sync_copy(zeros_hbm.at[pl.ds(r0, ROWS_PER_TILE)],
                    acc.at[pl.ds(r0, ROWS_PER_TILE)])

    @pl.when(s == NS - 1)
    def _zero_tail():
        pltpu.sync_copy(zeros_hbm.at[pl.ds(NS * ROWS_PER_TILE, ROWS_TAIL)],
                        acc.at[pl.ds(NS * ROWS_PER_TILE, ROWS_TAIL)])

    plsc.subcore_barrier()

    def scatter_wait(b):
        pltpu.make_async_copy(rows[b], acc.at[didx[b]], sem_s[b]).wait()
        pltpu.make_async_copy(rows[b], acc.at[didx[b]], sem_s[b]).wait()

    def do_chunk(eb, b, wait_prev):
        eb = pl.multiple_of(eb, 8)
        if wait_prev:
            scatter_wait(b)
        ci = pltpu.async_copy(src_hbm.at[pl.ds(eb, K)], sidx[b], sem_in[b])
        cj = pltpu.async_copy(dst_hbm.at[pl.ds(eb, K)], didx[b], sem_in[b])
        ci.wait()
        cj.wait()
        cg = pltpu.async_copy(x_hbm.at[sidx[b]], rows[b], sem_in[b])
        ca = pltpu.async_copy(attr_hbm.at[pl.ds(eb, K)], attr[b], sem_in[b])
        cg.wait()
        ca.wait()
        pltpu.async_copy(rows[b], acc.at[didx[b]], sem_s[b], add=True)
        pltpu.async_copy(attr[b], acc.at[didx[b]], sem_s[b], add=True)

    do_chunk(tile_base, 0, False)
    do_chunk(tile_base + K, 1, False)

    @pl.loop(0, (CHUNKS - 2) // 2)
    def _group(g):
        j = 2 + 2 * g
        do_chunk(tile_base + j * K, 0, True)
        do_chunk(tile_base + (j + 1) * K, 1, True)

    @pl.when(tid < EXTRA_TILES)
    def _extra():
        do_chunk(EXTRA_BASE + tid * K, 0, True)

    scatter_wait(1)
    scatter_wait(0)

    plsc.subcore_barrier()
    pltpu.sync_copy(acc.at[pl.ds(r0, ROWS_PER_TILE)],
                    out_hbm.at[c, pl.ds(r0, ROWS_PER_TILE)])

    @pl.when(s == NS - 1)
    def _write_tail():
        pltpu.sync_copy(acc.at[pl.ds(NS * ROWS_PER_TILE, ROWS_TAIL)],
                        out_hbm.at[c, pl.ds(NS * ROWS_PER_TILE, ROWS_TAIL)])


_sc_scatter = pl.kernel(
    _sc_body,
    out_type=jax.ShapeDtypeStruct((NC, N, D), jnp.float32),
    mesh=plsc.VectorSubcoreMesh(core_axis_name="c", subcore_axis_name="s"),
    scratch_types=[
        [pltpu.VMEM((K,), jnp.int32) for _ in range(2)],
        [pltpu.VMEM((K,), jnp.int32) for _ in range(2)],
        [pltpu.VMEM((K, D), jnp.float32) for _ in range(2)],
        [pltpu.VMEM((K, D), jnp.float32) for _ in range(2)],
        pltpu.VMEM_SHARED((N, D), jnp.float32),
        [pltpu.SemaphoreType.DMA for _ in range(2)],
        [pltpu.SemaphoreType.DMA for _ in range(2)],
    ],
)


def _combine_body(x_ref, p0_ref, p1_ref, o_ref):
    o_ref[...] = x_ref[...] + p0_ref[...] + p1_ref[...]


_combine = pl.pallas_call(
    _combine_body,
    grid=(10,),
    in_specs=[pl.BlockSpec((1000, D), lambda i: (i, 0))] * 3,
    out_specs=pl.BlockSpec((1000, D), lambda i: (i, 0)),
    out_shape=jax.ShapeDtypeStruct((N, D), jnp.float32),
)


@jax.jit
def kernel(x, edge_attr, edge_index):
    src = edge_index[0]
    dst = edge_index[1]
    zeros = jnp.zeros_like(x)
    partials = _sc_scatter(x, src, dst, edge_attr, zeros)
    return _combine(x, partials[0], partials[1])


# speedup vs baseline: 6.8509x; 1.2342x over previous
import jax
import jax.numpy as jnp
from jax import lax
from jax.experimental import pallas as pl
from jax.experimental.pallas import tpu as pltpu
from jax.experimental.pallas import tpu_sc as plsc

N = 10000
E = 320000
D = 128

NC = 2
NS = 16
NW = NC * NS
K = 64
CHUNKS = 156
EXTRA_BASE = NW * CHUNKS * K
EXTRA_TILES = (E - EXTRA_BASE) // K
ROWS_PER_TILE = 624
ROWS_TAIL = N - NS * ROWS_PER_TILE


def _sc_body(x_hbm, src_hbm, dst_hbm, attr_hbm, out_hbm,
             sidx, didx, rows, attr, acc,
             sem_idx, sem_in, sem_s):
    c = lax.axis_index("c")
    s = lax.axis_index("s")
    tid = c * NS + s
    tile_base = tid * (CHUNKS * K)

    zv = jnp.zeros((16,), jnp.float32)

    @pl.loop(0, K)
    def _zrow(r):
        for l in range(D // 16):
            rows[0][r, pl.ds(l * 16, 16)] = zv

    r0 = s * ROWS_PER_TILE
    for m in range(ROWS_PER_TILE // K):
        pltpu.sync_copy(rows[0], acc.at[pl.ds(r0 + m * K, K)])
    rem = ROWS_PER_TILE % K
    pltpu.sync_copy(rows[0].at[pl.ds(0, rem)],
                    acc.at[pl.ds(r0 + ROWS_PER_TILE - rem, rem)])

    @pl.when(s == NS - 1)
    def _zero_tail():
        pltpu.sync_copy(rows[0].at[pl.ds(0, ROWS_TAIL)],
                        acc.at[pl.ds(NS * ROWS_PER_TILE, ROWS_TAIL)])

    plsc.subcore_barrier()

    def scatter_wait(b):
        pltpu.make_async_copy(rows[b], acc.at[didx[0]], sem_s[b]).wait()
        pltpu.make_async_copy(rows[b], acc.at[didx[0]], sem_s[b]).wait()

    def issue_idx(eb, q):
        pltpu.async_copy(src_hbm.at[pl.ds(eb, K)], sidx[q], sem_idx[q])
        pltpu.async_copy(dst_hbm.at[pl.ds(eb, K)], didx[q], sem_idx[q])

    def idx_wait(q):
        pltpu.make_async_copy(src_hbm.at[pl.ds(0, K)], sidx[q], sem_idx[q]).wait()
        pltpu.make_async_copy(dst_hbm.at[pl.ds(0, K)], didx[q], sem_idx[q]).wait()

    def next_eb(j):
        nj = j + 2
        extra = jnp.where(tid < EXTRA_TILES, EXTRA_BASE + tid * K, 0)
        return jnp.where(nj < CHUNKS, tile_base + nj * K, extra)

    def do_chunk(eb, peb, b, q, qn, wait_prev, prefetch=True):
        eb = pl.multiple_of(eb, 8)
        if wait_prev:
            scatter_wait(b)
        if prefetch:
            issue_idx(pl.multiple_of(peb, 8), qn)
        idx_wait(q)
        cg = pltpu.async_copy(x_hbm.at[sidx[q]], rows[b], sem_in[b])
        ca = pltpu.async_copy(attr_hbm.at[pl.ds(eb, K)], attr[b], sem_in[b])
        cg.wait()
        ca.wait()
        pltpu.async_copy(rows[b], acc.at[didx[q]], sem_s[b], add=True)
        pltpu.async_copy(attr[b], acc.at[didx[q]], sem_s[b], add=True)

    issue_idx(tile_base, 0)
    issue_idx(tile_base + K, 1)
    for j in range(4):
        do_chunk(tile_base + j * K, tile_base + (j + 2) * K,
                 j % 2, j % 4, (j + 2) % 4, wait_prev=(j >= 2))

    @pl.loop(0, (CHUNKS - 4) // 4)
    def _group(g):
        j0 = 4 + 4 * g
        for u in range(4):
            j = j0 + u
            do_chunk(tile_base + j * K, next_eb(j), u % 2, u % 4,
                     (u + 2) % 4, wait_prev=True)

    @pl.when(tid < EXTRA_TILES)
    def _extra():
        do_chunk(EXTRA_BASE + tid * K, 0, 0, 0, 2,
                 wait_prev=True, prefetch=False)

    @pl.when(tid >= EXTRA_TILES)
    def _drain_idx0():
        idx_wait(0)

    idx_wait(1)
    scatter_wait(1)
    scatter_wait(0)

    plsc.subcore_barrier()
    pltpu.sync_copy(acc.at[pl.ds(r0, ROWS_PER_TILE)],
                    out_hbm.at[c, pl.ds(r0, ROWS_PER_TILE)])

    @pl.when(s == NS - 1)
    def _write_tail():
        pltpu.sync_copy(acc.at[pl.ds(NS * ROWS_PER_TILE, ROWS_TAIL)],
                        out_hbm.at[c, pl.ds(NS * ROWS_PER_TILE, ROWS_TAIL)])


_sc_scatter = pl.kernel(
    _sc_body,
    out_type=jax.ShapeDtypeStruct((NC, N, D), jnp.float32),
    mesh=plsc.VectorSubcoreMesh(core_axis_name="c", subcore_axis_name="s"),
    scratch_types=[
        [pltpu.VMEM((K,), jnp.int32) for _ in range(4)],
        [pltpu.VMEM((K,), jnp.int32) for _ in range(4)],
        [pltpu.VMEM((K, D), jnp.float32) for _ in range(2)],
        [pltpu.VMEM((K, D), jnp.float32) for _ in range(2)],
        pltpu.VMEM_SHARED((N, D), jnp.float32),
        [pltpu.SemaphoreType.DMA for _ in range(4)],
        [pltpu.SemaphoreType.DMA for _ in range(2)],
        [pltpu.SemaphoreType.DMA for _ in range(2)],
    ],
)


def _combine_body(x_ref, p0_ref, p1_ref, o_ref):
    o_ref[...] = x_ref[...] + p0_ref[...] + p1_ref[...]


_combine = pl.pallas_call(
    _combine_body,
    grid=(10,),
    in_specs=[pl.BlockSpec((1000, D), lambda i: (i, 0))] * 3,
    out_specs=pl.BlockSpec((1000, D), lambda i: (i, 0)),
    out_shape=jax.ShapeDtypeStruct((N, D), jnp.float32),
)


@jax.jit
def kernel(x, edge_attr, edge_index):
    partials = _sc_scatter(x, edge_index[0], edge_index[1], edge_attr)
    return _combine(x, partials[0], partials[1])


# speedup vs baseline: 8.6303x; 1.2597x over previous
import jax
import jax.numpy as jnp
from jax import lax
from jax.experimental import pallas as pl
from jax.experimental.pallas import tpu as pltpu
from jax.experimental.pallas import tpu_sc as plsc

N = 10000
E = 320000
D = 128

NC = 2
NS = 16
NW = NC * NS
K = 40
CHUNKS = E // (NW * K)
E_PER_TILE = CHUNKS * K
NB = 3
NQ = 6
ROWS_PER_TILE = 624
ROWS_TAIL = N - NS * ROWS_PER_TILE


def _sc_body(x_hbm, src_hbm, dst_hbm, attr_hbm, out_hbm,
             sidx, didx, rows, attr, acc,
             sem_idx, sem_in, sem_s):
    c = lax.axis_index("c")
    s = lax.axis_index("s")
    tid = c * NS + s
    tile_base = tid * E_PER_TILE

    zv = jnp.zeros((16,), jnp.float32)

    @pl.loop(0, K)
    def _zrow(r):
        for l in range(D // 16):
            rows[0][r, pl.ds(l * 16, 16)] = zv

    r0 = s * ROWS_PER_TILE
    for m in range(ROWS_PER_TILE // K):
        pltpu.sync_copy(rows[0], acc.at[pl.ds(r0 + m * K, K)])
    rem = ROWS_PER_TILE % K
    pltpu.sync_copy(rows[0].at[pl.ds(0, rem)],
                    acc.at[pl.ds(r0 + ROWS_PER_TILE - rem, rem)])

    @pl.when(s == NS - 1)
    def _zero_tail():
        pltpu.sync_copy(rows[0].at[pl.ds(0, ROWS_TAIL)],
                        acc.at[pl.ds(NS * ROWS_PER_TILE, ROWS_TAIL)])

    plsc.subcore_barrier()

    def issue_idx(eb, q):
        pltpu.async_copy(src_hbm.at[pl.ds(eb, K)], sidx[q], sem_idx[q])
        pltpu.async_copy(dst_hbm.at[pl.ds(eb, K)], didx[q], sem_idx[q])

    def idx_wait(q):
        pltpu.make_async_copy(src_hbm.at[pl.ds(0, K)], sidx[q], sem_idx[q]).wait()
        pltpu.make_async_copy(dst_hbm.at[pl.ds(0, K)], didx[q], sem_idx[q]).wait()

    def issue_in(eb, b, q):
        pltpu.async_copy(x_hbm.at[sidx[q]], rows[b], sem_in[b])
        pltpu.async_copy(attr_hbm.at[pl.ds(eb, K)], attr[b], sem_in[b])

    def in_wait(b, q):
        pltpu.make_async_copy(x_hbm.at[sidx[q]], rows[b], sem_in[b]).wait()
        pltpu.make_async_copy(attr_hbm.at[pl.ds(0, K)], attr[b], sem_in[b]).wait()

    def issue_scatter(b, q):
        pltpu.async_copy(rows[b], acc.at[didx[q]], sem_s[b], add=True)
        pltpu.async_copy(attr[b], acc.at[didx[q]], sem_s[b], add=True)

    def scatter_wait(b):
        pltpu.make_async_copy(rows[b], acc.at[didx[0]], sem_s[b]).wait()
        pltpu.make_async_copy(rows[b], acc.at[didx[0]], sem_s[b]).wait()

    def body(j, eb, peb, drain_prev_scatter, wait_prev_in):
        q, b = j % NQ, j % NB
        if drain_prev_scatter:
            scatter_wait(b)
        issue_idx(pl.multiple_of(peb, 8), (j + 2) % NQ)
        idx_wait(q)
        issue_in(pl.multiple_of(eb, 8), b, q)
        if wait_prev_in:
            in_wait((j - 1) % NB, (j - 1) % NQ)
            issue_scatter((j - 1) % NB, (j - 1) % NQ)

    issue_idx(tile_base, 0)
    issue_idx(tile_base + K, 1)
    body(0, tile_base, tile_base + 2 * K, False, False)
    body(1, tile_base + K, tile_base + 3 * K, False, True)
    body(2, tile_base + 2 * K, tile_base + 4 * K, False, True)
    body(3, tile_base + 3 * K, tile_base + 5 * K, True, True)

    @pl.loop(0, (CHUNKS - 4) // NQ)
    def _group(g):
        j0 = 4 + NQ * g
        for u in range(NQ):
            j = j0 + u
            nj = j + 2
            peb = jnp.where(nj < CHUNKS, tile_base + nj * K, 0)
            body(4 + u, tile_base + j * K, peb, True, True)

    last = CHUNKS - 1
    in_wait(last % NB, last % NQ)
    issue_scatter(last % NB, last % NQ)
    idx_wait((last + 1) % NQ)
    idx_wait((last + 2) % NQ)
    scatter_wait((last - 2) % NB)
    scatter_wait((last - 1) % NB)
    scatter_wait(last % NB)

    plsc.subcore_barrier()
    pltpu.sync_copy(acc.at[pl.ds(r0, ROWS_PER_TILE)],
                    out_hbm.at[c, pl.ds(r0, ROWS_PER_TILE)])

    @pl.when(s == NS - 1)
    def _write_tail():
        pltpu.sync_copy(acc.at[pl.ds(NS * ROWS_PER_TILE, ROWS_TAIL)],
                        out_hbm.at[c, pl.ds(NS * ROWS_PER_TILE, ROWS_TAIL)])


_sc_scatter = pl.kernel(
    _sc_body,
    out_type=jax.ShapeDtypeStruct((NC, N, D), jnp.float32),
    mesh=plsc.VectorSubcoreMesh(core_axis_name="c", subcore_axis_name="s"),
    scratch_types=[
        [pltpu.VMEM((K,), jnp.int32) for _ in range(NQ)],
        [pltpu.VMEM((K,), jnp.int32) for _ in range(NQ)],
        [pltpu.VMEM((K, D), jnp.float32) for _ in range(NB)],
        [pltpu.VMEM((K, D), jnp.float32) for _ in range(NB)],
        pltpu.VMEM_SHARED((N, D), jnp.float32),
        [pltpu.SemaphoreType.DMA for _ in range(NQ)],
        [pltpu.SemaphoreType.DMA for _ in range(NB)],
        [pltpu.SemaphoreType.DMA for _ in range(NB)],
    ],
)


def _combine_body(x_ref, p0_ref, p1_ref, o_ref):
    o_ref[...] = x_ref[...] + p0_ref[...] + p1_ref[...]


_combine = pl.pallas_call(
    _combine_body,
    grid=(10,),
    in_specs=[pl.BlockSpec((1000, D), lambda i: (i, 0))] * 3,
    out_specs=pl.BlockSpec((1000, D), lambda i: (i, 0)),
    out_shape=jax.ShapeDtypeStruct((N, D), jnp.float32),
)


@jax.jit
def kernel(x, edge_attr, edge_index):
    partials = _sc_scatter(x, edge_index[0], edge_index[1], edge_attr)
    return _combine(x, partials[0], partials[1])


# speedup vs baseline: 8.9090x; 1.0323x over previous
import jax
import jax.numpy as jnp
from jax import lax
from jax.experimental import pallas as pl
from jax.experimental.pallas import tpu as pltpu
from jax.experimental.pallas import tpu_sc as plsc

N = 10000
E = 320000
D = 128

NC = 2
NS = 16
NW = NC * NS
K = 40
CHUNKS = E // (NW * K)
E_PER_TILE = CHUNKS * K
NB = 3
NQ = 6
ROWS_PER_TILE = 624
ROWS_TAIL = N - NS * ROWS_PER_TILE


def _sc_body(x_hbm, src_hbm, dst_hbm, attr_hbm, out_hbm,
             sidx, didx, rows, attr, acc,
             sem_idx, sem_in, sem_s):
    c = lax.axis_index("c")
    s = lax.axis_index("s")
    tid = c * NS + s
    tile_base = tid * E_PER_TILE

    def issue_idx(eb, q):
        pltpu.async_copy(src_hbm.at[pl.ds(eb, K)], sidx[q], sem_idx[q])
        pltpu.async_copy(dst_hbm.at[pl.ds(eb, K)], didx[q], sem_idx[q])

    issue_idx(tile_base, 0)
    issue_idx(tile_base + K, 1)

    zv = jnp.zeros((16,), jnp.float32)

    @pl.loop(0, K)
    def _zrow(r):
        for l in range(D // 16):
            rows[0][r, pl.ds(l * 16, 16)] = zv

    r0 = s * ROWS_PER_TILE
    for m in range(ROWS_PER_TILE // K):
        pltpu.sync_copy(rows[0], acc.at[pl.ds(r0 + m * K, K)])
    rem = ROWS_PER_TILE % K
    pltpu.sync_copy(rows[0].at[pl.ds(0, rem)],
                    acc.at[pl.ds(r0 + ROWS_PER_TILE - rem, rem)])

    @pl.when(s == NS - 1)
    def _zero_tail():
        pltpu.sync_copy(rows[0].at[pl.ds(0, ROWS_TAIL)],
                        acc.at[pl.ds(NS * ROWS_PER_TILE, ROWS_TAIL)])

    plsc.subcore_barrier()

    def idx_wait(q):
        pltpu.make_async_copy(src_hbm.at[pl.ds(0, K)], sidx[q], sem_idx[q]).wait()
        pltpu.make_async_copy(dst_hbm.at[pl.ds(0, K)], didx[q], sem_idx[q]).wait()

    def issue_in(eb, b, q):
        pltpu.async_copy(x_hbm.at[sidx[q]], rows[b], sem_in[b])
        pltpu.async_copy(attr_hbm.at[pl.ds(eb, K)], attr[b], sem_in[b])

    def in_wait(b, q):
        pltpu.make_async_copy(x_hbm.at[sidx[q]], rows[b], sem_in[b]).wait()
        pltpu.make_async_copy(attr_hbm.at[pl.ds(0, K)], attr[b], sem_in[b]).wait()

    def issue_scatter(b, q):
        pltpu.async_copy(rows[b], acc.at[didx[q]], sem_s[b], add=True)
        pltpu.async_copy(attr[b], acc.at[didx[q]], sem_s[b], add=True)

    def scatter_wait(b):
        pltpu.make_async_copy(rows[b], acc.at[didx[0]], sem_s[b]).wait()
        pltpu.make_async_copy(rows[b], acc.at[didx[0]], sem_s[b]).wait()

    def body(j, eb, peb, drain_prev_scatter, wait_prev_in):
        q, b = j % NQ, j % NB
        if drain_prev_scatter:
            scatter_wait(b)
        issue_idx(pl.multiple_of(peb, 8), (j + 2) % NQ)
        idx_wait(q)
        issue_in(pl.multiple_of(eb, 8), b, q)
        if wait_prev_in:
            in_wait((j - 1) % NB, (j - 1) % NQ)
            issue_scatter((j - 1) % NB, (j - 1) % NQ)

    body(0, tile_base, tile_base + 2 * K, False, False)
    body(1, tile_base + K, tile_base + 3 * K, False, True)
    body(2, tile_base + 2 * K, tile_base + 4 * K, False, True)
    body(3, tile_base + 3 * K, tile_base + 5 * K, True, True)

    @pl.loop(0, (CHUNKS - 4) // NQ)
    def _group(g):
        j0 = 4 + NQ * g
        for u in range(NQ):
            j = j0 + u
            nj = j + 2
            peb = jnp.where(nj < CHUNKS, tile_base + nj * K, 0)
            body(4 + u, tile_base + j * K, peb, True, True)

    last = CHUNKS - 1
    in_wait(last % NB, last % NQ)
    issue_scatter(last % NB, last % NQ)
    idx_wait((last + 1) % NQ)
    idx_wait((last + 2) % NQ)
    scatter_wait((last - 2) % NB)
    scatter_wait((last - 1) % NB)
    scatter_wait(last % NB)

    plsc.subcore_barrier()
    pltpu.sync_copy(acc.at[pl.ds(r0, ROWS_PER_TILE)],
                    out_hbm.at[c, pl.ds(r0, ROWS_PER_TILE)])

    @pl.when(s == NS - 1)
    def _write_tail():
        pltpu.sync_copy(acc.at[pl.ds(NS * ROWS_PER_TILE, ROWS_TAIL)],
                        out_hbm.at[c, pl.ds(NS * ROWS_PER_TILE, ROWS_TAIL)])


_sc_scatter = pl.kernel(
    _sc_body,
    out_type=jax.ShapeDtypeStruct((NC, N, D), jnp.float32),
    mesh=plsc.VectorSubcoreMesh(core_axis_name="c", subcore_axis_name="s"),
    scratch_types=[
        [pltpu.VMEM((K,), jnp.int32) for _ in range(NQ)],
        [pltpu.VMEM((K,), jnp.int32) for _ in range(NQ)],
        [pltpu.VMEM((K, D), jnp.float32) for _ in range(NB)],
        [pltpu.VMEM((K, D), jnp.float32) for _ in range(NB)],
        pltpu.VMEM_SHARED((N, D), jnp.float32),
        [pltpu.SemaphoreType.DMA for _ in range(NQ)],
        [pltpu.SemaphoreType.DMA for _ in range(NB)],
        [pltpu.SemaphoreType.DMA for _ in range(NB)],
    ],
)


def _combine_body(x_ref, p_ref, o_ref):
    o_ref[...] = x_ref[...] + p_ref[0] + p_ref[1]


_combine = pl.pallas_call(
    _combine_body,
    grid=(10,),
    in_specs=[pl.BlockSpec((1000, D), lambda i: (i, 0)),
              pl.BlockSpec((NC, 1000, D), lambda i: (0, i, 0))],
    out_specs=pl.BlockSpec((1000, D), lambda i: (i, 0)),
    out_shape=jax.ShapeDtypeStruct((N, D), jnp.float32),
)


@jax.jit
def kernel(x, edge_attr, edge_index):
    partials = _sc_scatter(x, edge_index[0], edge_index[1], edge_attr)
    return _combine(x, partials)


# speedup vs baseline: 9.3618x; 1.0508x over previous
import jax
import jax.numpy as jnp
from jax import lax
from jax.experimental import pallas as pl
from jax.experimental.pallas import tpu as pltpu
from jax.experimental.pallas import tpu_sc as plsc

N = 10000
E = 320000
D = 128

NC = 2
NS = 16
NW = NC * NS
K = 40
CHUNKS = E // (NW * K)
E_PER_TILE = CHUNKS * K
NB = 3
NQ = 6
ROWS_PER_TILE = 624
ROWS_TAIL = N - NS * ROWS_PER_TILE


def _sc_body(x_hbm, ei_hbm, attr_hbm, out_hbm,
             sidx, didx, rows, attr, acc,
             sem_idx, sem_in, sem_s):
    c = lax.axis_index("c")
    s = lax.axis_index("s")
    tid = c * NS + s
    tile_base = tid * E_PER_TILE

    def issue_idx(eb, q):
        pltpu.async_copy(ei_hbm.at[pl.ds(eb, K)], sidx[q], sem_idx[q])
        pltpu.async_copy(ei_hbm.at[pl.ds(E + eb, K)], didx[q], sem_idx[q])

    issue_idx(tile_base, 0)
    issue_idx(tile_base + K, 1)

    zv = jnp.zeros((16,), jnp.float32)

    @pl.loop(0, K)
    def _zrow(r):
        for l in range(D // 16):
            rows[0][r, pl.ds(l * 16, 16)] = zv

    r0 = s * ROWS_PER_TILE
    for m in range(ROWS_PER_TILE // K):
        pltpu.sync_copy(rows[0], acc.at[pl.ds(r0 + m * K, K)])
    rem = ROWS_PER_TILE % K
    pltpu.sync_copy(rows[0].at[pl.ds(0, rem)],
                    acc.at[pl.ds(r0 + ROWS_PER_TILE - rem, rem)])

    @pl.when(s == NS - 1)
    def _zero_tail():
        pltpu.sync_copy(rows[0].at[pl.ds(0, ROWS_TAIL)],
                        acc.at[pl.ds(NS * ROWS_PER_TILE, ROWS_TAIL)])

    plsc.subcore_barrier()

    def idx_wait(q):
        pltpu.make_async_copy(ei_hbm.at[pl.ds(0, K)], sidx[q], sem_idx[q]).wait()
        pltpu.make_async_copy(ei_hbm.at[pl.ds(0, K)], didx[q], sem_idx[q]).wait()

    def issue_in(eb, b, q):
        pltpu.async_copy(x_hbm.at[sidx[q]], rows[b], sem_in[b])
        pltpu.async_copy(attr_hbm.at[pl.ds(eb, K)], attr[b], sem_in[b])

    def in_wait(b, q):
        pltpu.make_async_copy(x_hbm.at[sidx[q]], rows[b], sem_in[b]).wait()
        pltpu.make_async_copy(attr_hbm.at[pl.ds(0, K)], attr[b], sem_in[b]).wait()

    def issue_scatter(b, q):
        pltpu.async_copy(rows[b], acc.at[didx[q]], sem_s[b], add=True)
        pltpu.async_copy(attr[b], acc.at[didx[q]], sem_s[b], add=True)

    def scatter_wait(b):
        pltpu.make_async_copy(rows[b], acc.at[didx[0]], sem_s[b]).wait()
        pltpu.make_async_copy(rows[b], acc.at[didx[0]], sem_s[b]).wait()

    def body(j, eb, peb, drain_prev_scatter, wait_prev_in):
        q, b = j % NQ, j % NB
        if drain_prev_scatter:
            scatter_wait(b)
        issue_idx(pl.multiple_of(peb, 8), (j + 2) % NQ)
        idx_wait(q)
        issue_in(pl.multiple_of(eb, 8), b, q)
        if wait_prev_in:
            in_wait((j - 1) % NB, (j - 1) % NQ)
            issue_scatter((j - 1) % NB, (j - 1) % NQ)

    body(0, tile_base, tile_base + 2 * K, False, False)
    body(1, tile_base + K, tile_base + 3 * K, False, True)
    body(2, tile_base + 2 * K, tile_base + 4 * K, False, True)
    body(3, tile_base + 3 * K, tile_base + 5 * K, True, True)

    @pl.loop(0, (CHUNKS - 4) // NQ)
    def _group(g):
        j0 = 4 + NQ * g
        for u in range(NQ):
            j = j0 + u
            nj = j + 2
            peb = jnp.where(nj < CHUNKS, tile_base + nj * K, 0)
            body(4 + u, tile_base + j * K, peb, True, True)

    last = CHUNKS - 1
    in_wait(last % NB, last % NQ)
    issue_scatter(last % NB, last % NQ)
    idx_wait((last + 1) % NQ)
    idx_wait((last + 2) % NQ)
    scatter_wait((last - 2) % NB)
    scatter_wait((last - 1) % NB)
    scatter_wait(last % NB)

    plsc.subcore_barrier()
    pltpu.sync_copy(acc.at[pl.ds(r0, ROWS_PER_TILE)],
                    out_hbm.at[c, pl.ds(r0, ROWS_PER_TILE)])

    @pl.when(s == NS - 1)
    def _write_tail():
        pltpu.sync_copy(acc.at[pl.ds(NS * ROWS_PER_TILE, ROWS_TAIL)],
                        out_hbm.at[c, pl.ds(NS * ROWS_PER_TILE, ROWS_TAIL)])


_sc_scatter = pl.kernel(
    _sc_body,
    out_type=jax.ShapeDtypeStruct((NC, N, D), jnp.float32),
    mesh=plsc.VectorSubcoreMesh(core_axis_name="c", subcore_axis_name="s"),
    scratch_types=[
        [pltpu.VMEM((K,), jnp.int32) for _ in range(NQ)],
        [pltpu.VMEM((K,), jnp.int32) for _ in range(NQ)],
        [pltpu.VMEM((K, D), jnp.float32) for _ in range(NB)],
        [pltpu.VMEM((K, D), jnp.float32) for _ in range(NB)],
        pltpu.VMEM_SHARED((N, D), jnp.float32),
        [pltpu.SemaphoreType.DMA for _ in range(NQ)],
        [pltpu.SemaphoreType.DMA for _ in range(NB)],
        [pltpu.SemaphoreType.DMA for _ in range(NB)],
    ],
)


def _combine_body(x_ref, p_ref, o_ref):
    o_ref[...] = x_ref[...] + p_ref[0] + p_ref[1]


_combine = pl.pallas_call(
    _combine_body,
    grid=(10,),
    in_specs=[pl.BlockSpec((1000, D), lambda i: (i, 0)),
              pl.BlockSpec((NC, 1000, D), lambda i: (0, i, 0))],
    out_specs=pl.BlockSpec((1000, D), lambda i: (i, 0)),
    out_shape=jax.ShapeDtypeStruct((N, D), jnp.float32),
)


@jax.jit
def kernel(x, edge_attr, edge_index):
    partials = _sc_scatter(x, edge_index.reshape(2 * E), edge_attr)
    return _combine(x, partials)
